# baseline (speedup 1.0000x reference)
import jax
import jax.numpy as jnp
import numpy as np
from jax import lax
from jax.experimental import pallas as pl

_LEGAL_ISO = np.array(
    [50, 64, 80, 100, 125, 160, 200, 250, 320, 400, 500, 640, 800, 1000,
     1250, 1600, 2000, 2500, 3200, 4000, 5000, 6400, 8000, 10000, 12800,
     16000, 20000, 25600, 32000, 40000, 51200], dtype=np.float32)
_N_TBL = 31


def _compute_scale(params_row):
    tbl = params_row[0:1, 0:_N_TBL]
    cam_row = params_row[0:1, _N_TBL:2 * _N_TBL]
    gain = params_row[0, 2 * _N_TBL]
    iso_v = params_row[0, 2 * _N_TBL + 1]
    lt = (tbl < iso_v).astype(jnp.int32)
    le = (tbl <= iso_v).astype(jnp.int32)
    l_idx = jnp.minimum(jnp.sum(lt), _N_TBL - 1)
    r_idx = jnp.minimum(jnp.sum(le), _N_TBL - 1)
    iota = lax.broadcasted_iota(jnp.int32, (1, _N_TBL), 1)
    sel_l = (iota == l_idx)
    sel_r = (iota == r_idx)
    zf = jnp.zeros((1, _N_TBL), jnp.float32)
    iso_l = jnp.sum(jnp.where(sel_l, tbl, zf))
    iso_r = jnp.sum(jnp.where(sel_r, tbl, zf))
    cam_l = jnp.exp(jnp.sum(jnp.where(sel_l, cam_row, zf)))
    cam_r = jnp.exp(jnp.sum(jnp.where(sel_r, cam_row, zf)))
    denom = iso_r - iso_l
    safe_denom = jnp.where(denom != 0, denom, jnp.float32(1.0))
    cam = jnp.where(denom != 0,
                    ((iso_v - iso_l) * cam_r + (iso_r - iso_v) * cam_l) / safe_denom,
                    cam_l)
    return jnp.exp(cam * gain) * iso_v


def _gain_iso_kernel(params_ref, x_ref, z_ref, logdet_ref, *, n_chw, n_batch):
    scale = _compute_scale(params_ref[...])
    inv = jnp.float32(1.0) / scale
    z_ref[...] = x_ref[...] * inv
    val = -jnp.float32(n_chw) * jnp.log(scale)
    logdet_ref[...] = jnp.zeros((1, n_batch), jnp.float32) + val


def kernel(x, cam_param, gain_params, iso):
    B, C, H, W = x.shape
    n_chw = C * H * W
    total = B * n_chw
    ncols = 4096
    nrows = total // ncols
    block_rows = 256
    grid = nrows // block_rows

    iso_f = jnp.asarray(iso, jnp.float32)
    gain_f = jnp.asarray(gain_params, jnp.float32)
    params_row = jnp.concatenate(
        [jnp.asarray(_LEGAL_ISO), cam_param.astype(jnp.float32),
         gain_f[None], iso_f[None]]).reshape(1, 2 * _N_TBL + 2)

    x2d = x.reshape(nrows, ncols)

    import functools
    body = functools.partial(_gain_iso_kernel, n_chw=n_chw, n_batch=B)
    z2d, logdet = pl.pallas_call(
        body,
        grid=(grid,),
        in_specs=[
            pl.BlockSpec((1, 2 * _N_TBL + 2), lambda i: (0, 0)),
            pl.BlockSpec((block_rows, ncols), lambda i: (i, 0)),
        ],
        out_specs=[
            pl.BlockSpec((block_rows, ncols), lambda i: (i, 0)),
            pl.BlockSpec((1, B), lambda i: (0, 0)),
        ],
        out_shape=[
            jax.ShapeDtypeStruct((nrows, ncols), jnp.float32),
            jax.ShapeDtypeStruct((1, B), jnp.float32),
        ],
    )(params_row, x2d)

    return z2d.reshape(B, C, H, W), logdet.reshape(B)
